# baseline (speedup 1.0000x reference)
import functools

import jax
import jax.numpy as jnp
import numpy as np
from jax import lax
from jax.experimental import pallas as pl
from jax.experimental.pallas import tpu as pltpu
from jax.experimental.pallas import tpu_sc as plsc

_B, _T, _M, _D = 16, 2048, 9, 128
_V = _M - 1
_R = _V // 2
_P = _B * _T
_NW = 32
_TPW = _P // _NW
_CHUNK = 16
_ROW_CH = 128
_NRCH = _TPW * _R // _ROW_CH
_NTB = _TPW // _ROW_CH


def _body(table, noise, pad, fcst,
          out_data, out_remain, out_masked, out_revert, out_mask9, out_mask5,
          noise_v, pad_v, fcst_v, remain_v, masked_v, revert_v,
          mask9_v, mask5_v, idxlist_v, buf0, buf1, buf2, buf3,
          gsem0, gsem1, gsem2, gsem3, wsem0, wsem1, wsem2, wsem3, msem, csem):
    wid = lax.axis_index("s") * 2 + lax.axis_index("c")
    base = wid * _TPW
    b_id = wid // 2
    t0 = (wid % 2) * _TPW

    cdescs = [
        pltpu.async_copy(
            table.at[pl.ds(base + tb * _ROW_CH, _ROW_CH)],
            out_data.at[pl.ds(b_id * (_R + 1) * _T + t0 + tb * _ROW_CH,
                              _ROW_CH)],
            csem)
        for tb in range(_NTB)
    ]

    pltpu.sync_copy(noise.at[pl.ds(base * _V, _TPW * _V)], noise_v)
    pltpu.sync_copy(pad.at[pl.ds(base, _TPW)], pad_v)
    pltpu.sync_copy(fcst.at[pl.ds(base, _TPW)], fcst_v)

    lane = lax.iota(jnp.int32, _CHUNK)

    def chunk(i, carry):
        p_l = i * _CHUNK + lane
        p_g = base + p_l
        n = [plsc.load_gather(noise_v, [p_l * _V + m]) for m in range(_V)]
        pd = plsc.load_gather(pad_v, [p_l])
        fc = plsc.load_gather(fcst_v, [p_l])

        rev = [jnp.zeros((_CHUNK,), jnp.int32) for _ in range(_V)]
        for a in range(_V):
            for b in range(a + 1, _V):
                pre = (n[a] <= n[b]).astype(jnp.int32)
                rev[b] = rev[b] + pre
                rev[a] = rev[a] + (1 - pre)

        tb_l = p_l >> 7
        col = p_l & 127
        for m in range(_V):
            keep = rev[m] < _R
            rk = jnp.minimum(rev[m], _R - 1)
            mk = jnp.maximum(rev[m] - _R, 0)
            mval = jnp.full((_CHUNK,), m, jnp.int32)
            plsc.store_scatter(remain_v, [rk * _TPW + p_l], mval, mask=keep)
            plsc.store_scatter(masked_v, [mk * _TPW + p_l], mval, mask=~keep)
            plsc.store_scatter(revert_v, [m * _TPW + p_l], rev[m])
            plsc.store_scatter(idxlist_v, [tb_l * _R + rk, col],
                               p_g + (m + 1) * _P, mask=keep)

        for l in range(_M):
            plsc.store_scatter(mask9_v, [l * _TPW + p_l], fc if l == 1 else pd)
        for l in range(_R + 1):
            plsc.store_scatter(mask5_v, [l * _TPW + p_l], pd)
        r0 = jnp.minimum(rev[0], _R - 1)
        plsc.store_scatter(mask5_v, [(1 + r0) * _TPW + p_l], fc,
                          mask=rev[0] < _R)
        return carry

    done = [0]

    def ensure(nblocks):
        while done[0] < nblocks:
            tb = done[0]
            lax.fori_loop(tb * (_ROW_CH // _CHUNK), (tb + 1) * (_ROW_CH // _CHUNK),
                          chunk, 0)
            done[0] += 1

    bufs = (buf0, buf1, buf2, buf3)
    gsems = (gsem0, gsem1, gsem2, gsem3)
    wsems = (wsem0, wsem1, wsem2, wsem3)

    def outrow(k):
        j = k % _R + 1
        return (b_id * (_R + 1) + j) * _T + t0 + (k // _R) * _ROW_CH

    def gather(k, s):
        return pltpu.async_copy(table.at[idxlist_v.at[k]], bufs[s], gsems[s])

    gd = [None] * 4
    wd = [None] * 4
    ensure(1)
    gd[0] = gather(0, 0)
    gd[1] = gather(1, 1)
    for k in range(_NRCH):
        s = k % 4
        gd[s].wait()
        wd[s] = pltpu.async_copy(bufs[s],
                                 out_data.at[pl.ds(outrow(k), _ROW_CH)],
                                 wsems[s])
        if k + 2 < _NRCH:
            ensure((k + 2) // _R + 1)
            s2 = (k + 2) % 4
            if wd[s2] is not None:
                wd[s2].wait()
            gd[s2] = gather(k + 2, s2)

    mdescs = [
        pltpu.async_copy(remain_v,
                         out_remain.at[pl.ds(wid * _TPW * _R, _TPW * _R)],
                         msem),
        pltpu.async_copy(masked_v,
                         out_masked.at[pl.ds(wid * _TPW * _R, _TPW * _R)],
                         msem),
        pltpu.async_copy(revert_v,
                         out_revert.at[pl.ds(wid * _TPW * _V, _TPW * _V)],
                         msem),
        pltpu.async_copy(mask9_v,
                         out_mask9.at[pl.ds(wid * _TPW * _M, _TPW * _M)],
                         msem),
        pltpu.async_copy(mask5_v,
                         out_mask5.at[pl.ds(wid * _TPW * (_R + 1),
                                            _TPW * (_R + 1))],
                         msem),
    ]
    for s in range(4):
        if wd[s] is not None:
            wd[s].wait()
    for d in cdescs:
        d.wait()
    for d in mdescs:
        d.wait()


_mesh = plsc.VectorSubcoreMesh(core_axis_name="c", subcore_axis_name="s",
                               num_cores=2, num_subcores=16)

_sc_call = functools.partial(
    pl.kernel,
    out_type=(
        jax.ShapeDtypeStruct((_P * (_R + 1), _D), jnp.float32),
        jax.ShapeDtypeStruct((_P * _R,), jnp.int32),
        jax.ShapeDtypeStruct((_P * _R,), jnp.int32),
        jax.ShapeDtypeStruct((_P * _V,), jnp.int32),
        jax.ShapeDtypeStruct((_P * _M,), jnp.int32),
        jax.ShapeDtypeStruct((_P * (_R + 1),), jnp.int32),
    ),
    mesh=_mesh,
    compiler_params=pltpu.CompilerParams(needs_layout_passes=False),
    scratch_types=(
        pltpu.VMEM((_TPW * _V,), jnp.float32),
        pltpu.VMEM((_TPW,), jnp.int32),
        pltpu.VMEM((_TPW,), jnp.int32),
        pltpu.VMEM((_TPW * _R,), jnp.int32),
        pltpu.VMEM((_TPW * _R,), jnp.int32),
        pltpu.VMEM((_TPW * _V,), jnp.int32),
        pltpu.VMEM((_TPW * _M,), jnp.int32),
        pltpu.VMEM((_TPW * (_R + 1),), jnp.int32),
        pltpu.VMEM((_NRCH, _ROW_CH), jnp.int32),
        pltpu.VMEM((_ROW_CH, _D), jnp.float32),
        pltpu.VMEM((_ROW_CH, _D), jnp.float32),
        pltpu.VMEM((_ROW_CH, _D), jnp.float32),
        pltpu.VMEM((_ROW_CH, _D), jnp.float32),
        pltpu.SemaphoreType.DMA,
        pltpu.SemaphoreType.DMA,
        pltpu.SemaphoreType.DMA,
        pltpu.SemaphoreType.DMA,
        pltpu.SemaphoreType.DMA,
        pltpu.SemaphoreType.DMA,
        pltpu.SemaphoreType.DMA,
        pltpu.SemaphoreType.DMA,
        pltpu.SemaphoreType.DMA,
        pltpu.SemaphoreType.DMA,
    ),
)(_body)


def _np_threefry_uniform(seed, size):
    def rotl(x, d):
        return ((x << np.uint32(d)) | (x >> np.uint32(32 - d))).astype(np.uint32)

    ks0 = np.uint32(seed >> 32)
    ks1 = np.uint32(seed & 0xFFFFFFFF)
    ks = (ks0, ks1, np.uint32(np.uint32(0x1BD11BDA) ^ ks0 ^ ks1))
    rot = ((13, 15, 26, 6), (17, 29, 16, 24))
    x0 = (np.zeros(size, np.uint32) + ks0).astype(np.uint32)
    x1 = (np.arange(size, dtype=np.uint32) + ks1).astype(np.uint32)
    for i in range(5):
        for d in rot[i % 2]:
            x0 = (x0 + x1).astype(np.uint32)
            x1 = (rotl(x1, d) ^ x0).astype(np.uint32)
        x0 = (x0 + ks[(i + 1) % 3]).astype(np.uint32)
        x1 = (x1 + ks[(i + 2) % 3] + np.uint32(i + 1)).astype(np.uint32)
    bits = x0 ^ x1
    fb = ((bits >> np.uint32(9)) | np.uint32(0x3F800000)).view(np.float32)
    return np.maximum(fb - np.float32(1.0), np.float32(0.0))


_NOISE = _np_threefry_uniform(42, _P * _V).reshape(_B, _T, _V)


def kernel(temporal_data, temporal_padding_mask, target_fcst_mask):
    noise = jnp.asarray(_NOISE)
    table = temporal_data.reshape(_M * _B * _T, _D)
    data, remain, masked, revert, m9, m5 = _sc_call(
        table,
        noise.reshape(_P * _V),
        temporal_padding_mask.reshape(_P).astype(jnp.int32),
        target_fcst_mask.reshape(_P).astype(jnp.int32),
    )

    def _unplanar(x, k):
        return x.reshape(_NW, k, _TPW).transpose(0, 2, 1).reshape(_B, _T, k)

    return (data.reshape(_B, _R + 1, _T, _D).transpose(0, 2, 1, 3),
            _unplanar(remain, _R),
            _unplanar(masked, _R),
            _unplanar(revert, _V),
            _unplanar(m5, _R + 1) != 0,
            _unplanar(m9, _M) != 0)


# speedup vs baseline: 5.7667x; 5.7667x over previous
import functools

import jax
import jax.numpy as jnp
import numpy as np
from jax import lax
from jax.experimental import pallas as pl
from jax.experimental.pallas import tpu as pltpu
from jax.experimental.pallas import tpu_sc as plsc

_B, _T, _M, _D = 16, 2048, 9, 128
_V = _M - 1
_R = _V // 2
_P = _B * _T
_NW = 32
_TPW = _P // _NW
_CHUNK = 16
_ROW_CH = 128
_NRCH = _TPW * (_R + 1) // _ROW_CH
_NTB = _TPW // _ROW_CH


def _body(table, noise, pad, fcst,
          out_data, out_remain, out_masked, out_revert, out_mask9, out_mask5,
          noise_v, pad_v, fcst_v, remain_v, masked_v, revert_v,
          mask9_v, mask5_v, idxlist_v, buf0, buf1, buf2, buf3,
          gsem0, gsem1, gsem2, gsem3, wsem0, wsem1, wsem2, wsem3, msem):
    wid = lax.axis_index("s") * 2 + lax.axis_index("c")
    base = wid * _TPW
    b_id = wid // 2
    t0 = (wid % 2) * _TPW


    pltpu.sync_copy(noise.at[pl.ds(base * _V, _TPW * _V)], noise_v)
    pltpu.sync_copy(pad.at[pl.ds(base, _TPW)], pad_v)
    pltpu.sync_copy(fcst.at[pl.ds(base, _TPW)], fcst_v)

    lane = lax.iota(jnp.int32, _CHUNK)

    def chunk(i, carry):
        p_l = i * _CHUNK + lane
        p_g = base + p_l
        n = [plsc.load_gather(noise_v, [p_l * _V + m]) for m in range(_V)]
        pd = plsc.load_gather(pad_v, [p_l])
        fc = plsc.load_gather(fcst_v, [p_l])

        rev = [jnp.zeros((_CHUNK,), jnp.int32) for _ in range(_V)]
        for a in range(_V):
            for b in range(a + 1, _V):
                pre = (n[a] <= n[b]).astype(jnp.int32)
                rev[b] = rev[b] + pre
                rev[a] = rev[a] + (1 - pre)

        tb_l = p_l >> 7
        col = p_l & 127
        plsc.store_scatter(idxlist_v, [tb_l * (_R + 1), col], p_g)
        for m in range(_V):
            keep = rev[m] < _R
            rk = jnp.minimum(rev[m], _R - 1)
            mk = jnp.maximum(rev[m] - _R, 0)
            mval = jnp.full((_CHUNK,), m, jnp.int32)
            plsc.store_scatter(remain_v, [rk * _TPW + p_l], mval, mask=keep)
            plsc.store_scatter(masked_v, [mk * _TPW + p_l], mval, mask=~keep)
            plsc.store_scatter(revert_v, [m * _TPW + p_l], rev[m])
            plsc.store_scatter(idxlist_v, [tb_l * (_R + 1) + 1 + rk, col],
                               p_g + (m + 1) * _P, mask=keep)

        for l in range(_M):
            plsc.store_scatter(mask9_v, [l * _TPW + p_l], fc if l == 1 else pd)
        for l in range(_R + 1):
            plsc.store_scatter(mask5_v, [l * _TPW + p_l], pd)
        r0 = jnp.minimum(rev[0], _R - 1)
        plsc.store_scatter(mask5_v, [(1 + r0) * _TPW + p_l], fc,
                          mask=rev[0] < _R)
        return carry

    done = [0]

    def ensure(nblocks):
        while done[0] < nblocks:
            tb = done[0]
            lax.fori_loop(tb * (_ROW_CH // _CHUNK), (tb + 1) * (_ROW_CH // _CHUNK),
                          chunk, 0)
            done[0] += 1

    bufs = (buf0, buf1, buf2, buf3)
    gsems = (gsem0, gsem1, gsem2, gsem3)
    wsems = (wsem0, wsem1, wsem2, wsem3)

    def outrow(k):
        j = k % (_R + 1)
        return (b_id * (_R + 1) + j) * _T + t0 + (k // (_R + 1)) * _ROW_CH

    def gather(k, s):
        return pltpu.async_copy(table.at[idxlist_v.at[k]], bufs[s], gsems[s])

    gd = [None] * 4
    wd = [None] * 4
    ensure(1)
    gd[0] = gather(0, 0)
    gd[1] = gather(1, 1)
    for k in range(_NRCH):
        s = k % 4
        gd[s].wait()
        wd[s] = pltpu.async_copy(bufs[s],
                                 out_data.at[pl.ds(outrow(k), _ROW_CH)],
                                 wsems[s])
        if k + 2 < _NRCH:
            ensure((k + 2) // (_R + 1) + 1)
            s2 = (k + 2) % 4
            if wd[s2] is not None:
                wd[s2].wait()
            gd[s2] = gather(k + 2, s2)

    mdescs = [
        pltpu.async_copy(remain_v,
                         out_remain.at[pl.ds(wid * _TPW * _R, _TPW * _R)],
                         msem),
        pltpu.async_copy(masked_v,
                         out_masked.at[pl.ds(wid * _TPW * _R, _TPW * _R)],
                         msem),
        pltpu.async_copy(revert_v,
                         out_revert.at[pl.ds(wid * _TPW * _V, _TPW * _V)],
                         msem),
        pltpu.async_copy(mask9_v,
                         out_mask9.at[pl.ds(wid * _TPW * _M, _TPW * _M)],
                         msem),
        pltpu.async_copy(mask5_v,
                         out_mask5.at[pl.ds(wid * _TPW * (_R + 1),
                                            _TPW * (_R + 1))],
                         msem),
    ]
    for s in range(4):
        if wd[s] is not None:
            wd[s].wait()
    for d in mdescs:
        d.wait()


_mesh = plsc.VectorSubcoreMesh(core_axis_name="c", subcore_axis_name="s",
                               num_cores=2, num_subcores=16)

_sc_call = functools.partial(
    pl.kernel,
    out_type=(
        jax.ShapeDtypeStruct((_P * (_R + 1), _D), jnp.float32),
        jax.ShapeDtypeStruct((_P * _R,), jnp.int32),
        jax.ShapeDtypeStruct((_P * _R,), jnp.int32),
        jax.ShapeDtypeStruct((_P * _V,), jnp.int32),
        jax.ShapeDtypeStruct((_P * _M,), jnp.int32),
        jax.ShapeDtypeStruct((_P * (_R + 1),), jnp.int32),
    ),
    mesh=_mesh,
    compiler_params=pltpu.CompilerParams(needs_layout_passes=False),
    scratch_types=(
        pltpu.VMEM((_TPW * _V,), jnp.float32),
        pltpu.VMEM((_TPW,), jnp.int32),
        pltpu.VMEM((_TPW,), jnp.int32),
        pltpu.VMEM((_TPW * _R,), jnp.int32),
        pltpu.VMEM((_TPW * _R,), jnp.int32),
        pltpu.VMEM((_TPW * _V,), jnp.int32),
        pltpu.VMEM((_TPW * _M,), jnp.int32),
        pltpu.VMEM((_TPW * (_R + 1),), jnp.int32),
        pltpu.VMEM((_NRCH, _ROW_CH), jnp.int32),
        pltpu.VMEM((_ROW_CH, _D), jnp.float32),
        pltpu.VMEM((_ROW_CH, _D), jnp.float32),
        pltpu.VMEM((_ROW_CH, _D), jnp.float32),
        pltpu.VMEM((_ROW_CH, _D), jnp.float32),
        pltpu.SemaphoreType.DMA,
        pltpu.SemaphoreType.DMA,
        pltpu.SemaphoreType.DMA,
        pltpu.SemaphoreType.DMA,
        pltpu.SemaphoreType.DMA,
        pltpu.SemaphoreType.DMA,
        pltpu.SemaphoreType.DMA,
        pltpu.SemaphoreType.DMA,
        pltpu.SemaphoreType.DMA,
    ),
)(_body)


def _np_threefry_uniform(seed, size):
    def rotl(x, d):
        return ((x << np.uint32(d)) | (x >> np.uint32(32 - d))).astype(np.uint32)

    ks0 = np.uint32(seed >> 32)
    ks1 = np.uint32(seed & 0xFFFFFFFF)
    ks = (ks0, ks1, np.uint32(np.uint32(0x1BD11BDA) ^ ks0 ^ ks1))
    rot = ((13, 15, 26, 6), (17, 29, 16, 24))
    x0 = (np.zeros(size, np.uint32) + ks0).astype(np.uint32)
    x1 = (np.arange(size, dtype=np.uint32) + ks1).astype(np.uint32)
    for i in range(5):
        for d in rot[i % 2]:
            x0 = (x0 + x1).astype(np.uint32)
            x1 = (rotl(x1, d) ^ x0).astype(np.uint32)
        x0 = (x0 + ks[(i + 1) % 3]).astype(np.uint32)
        x1 = (x1 + ks[(i + 2) % 3] + np.uint32(i + 1)).astype(np.uint32)
    bits = x0 ^ x1
    fb = ((bits >> np.uint32(9)) | np.uint32(0x3F800000)).view(np.float32)
    return np.maximum(fb - np.float32(1.0), np.float32(0.0))


_NOISE = _np_threefry_uniform(42, _P * _V).reshape(_B, _T, _V)


def kernel(temporal_data, temporal_padding_mask, target_fcst_mask):
    noise = jnp.asarray(_NOISE)
    table = temporal_data.reshape(_M * _B * _T, _D)
    data, remain, masked, revert, m9, m5 = _sc_call(
        table,
        noise.reshape(_P * _V),
        temporal_padding_mask.reshape(_P).astype(jnp.int32),
        target_fcst_mask.reshape(_P).astype(jnp.int32),
    )

    def _unplanar(x, k):
        return x.reshape(_NW, k, _TPW).transpose(0, 2, 1).reshape(_B, _T, k)

    return (data.reshape(_B, _R + 1, _T, _D).transpose(0, 2, 1, 3),
            _unplanar(remain, _R),
            _unplanar(masked, _R),
            _unplanar(revert, _V),
            _unplanar(m5, _R + 1) != 0,
            _unplanar(m9, _M) != 0)


# speedup vs baseline: 5.8183x; 1.0090x over previous
import functools

import jax
import jax.numpy as jnp
import numpy as np
from jax import lax
from jax.experimental import pallas as pl
from jax.experimental.pallas import tpu as pltpu
from jax.experimental.pallas import tpu_sc as plsc

_B, _T, _M, _D = 16, 2048, 9, 128
_V = _M - 1
_R = _V // 2
_P = _B * _T
_NW = 32
_TPW = _P // _NW
_CHUNK = 16
_ROW_CH = 128
_NRCH = _TPW * (_R + 1) // _ROW_CH
_NTB = _TPW // _ROW_CH


def _body(table, noise, pad, fcst,
          out_data, out_remain, out_masked, out_revert, out_mask9, out_mask5,
          noise_v, pad_v, fcst_v, remain_v, masked_v, revert_v,
          mask9_v, mask5_v, idxlist_v, buf0, buf1, buf2, buf3, buf4,
          gsem0, gsem1, gsem2, gsem3, gsem4,
          wsem0, wsem1, wsem2, wsem3, wsem4, msem):
    wid = lax.axis_index("s") * 2 + lax.axis_index("c")
    base = wid * _TPW
    b_id = wid // 2
    t0 = (wid % 2) * _TPW


    pltpu.sync_copy(noise.at[pl.ds(base * _V, _TPW * _V)], noise_v)
    pltpu.sync_copy(pad.at[pl.ds(base, _TPW)], pad_v)
    pltpu.sync_copy(fcst.at[pl.ds(base, _TPW)], fcst_v)

    lane = lax.iota(jnp.int32, _CHUNK)

    def chunk(i, carry):
        p_l = i * _CHUNK + lane
        p_g = base + p_l
        n = [plsc.load_gather(noise_v, [p_l * _V + m]) for m in range(_V)]
        pd = plsc.load_gather(pad_v, [p_l])
        fc = plsc.load_gather(fcst_v, [p_l])

        rev = [jnp.zeros((_CHUNK,), jnp.int32) for _ in range(_V)]
        for a in range(_V):
            for b in range(a + 1, _V):
                pre = (n[a] <= n[b]).astype(jnp.int32)
                rev[b] = rev[b] + pre
                rev[a] = rev[a] + (1 - pre)

        tb_l = p_l >> 7
        col = p_l & 127
        plsc.store_scatter(idxlist_v, [tb_l * (_R + 1), col], p_g)
        for m in range(_V):
            keep = rev[m] < _R
            rk = jnp.minimum(rev[m], _R - 1)
            mk = jnp.maximum(rev[m] - _R, 0)
            mval = jnp.full((_CHUNK,), m, jnp.int32)
            plsc.store_scatter(remain_v, [rk * _TPW + p_l], mval, mask=keep)
            plsc.store_scatter(masked_v, [mk * _TPW + p_l], mval, mask=~keep)
            plsc.store_scatter(revert_v, [m * _TPW + p_l], rev[m])
            plsc.store_scatter(idxlist_v, [tb_l * (_R + 1) + 1 + rk, col],
                               p_g + (m + 1) * _P, mask=keep)

        for l in range(_M):
            plsc.store_scatter(mask9_v, [l * _TPW + p_l], fc if l == 1 else pd)
        for l in range(_R + 1):
            plsc.store_scatter(mask5_v, [l * _TPW + p_l], pd)
        r0 = jnp.minimum(rev[0], _R - 1)
        plsc.store_scatter(mask5_v, [(1 + r0) * _TPW + p_l], fc,
                          mask=rev[0] < _R)
        return carry

    done = [0]

    def ensure(nblocks):
        while done[0] < nblocks:
            tb = done[0]
            lax.fori_loop(tb * (_ROW_CH // _CHUNK), (tb + 1) * (_ROW_CH // _CHUNK),
                          chunk, 0)
            done[0] += 1

    bufs = (buf0, buf1, buf2, buf3, buf4)
    gsems = (gsem0, gsem1, gsem2, gsem3, gsem4)
    wsems = (wsem0, wsem1, wsem2, wsem3, wsem4)
    _NB = 5

    def outrow(k):
        j = k % (_R + 1)
        return (b_id * (_R + 1) + j) * _T + t0 + (k // (_R + 1)) * _ROW_CH

    def gather(k, s):
        return pltpu.async_copy(table.at[idxlist_v.at[k]], bufs[s], gsems[s])

    _LA = _NB - 2
    gd = [None] * _NB
    wd = [None] * _NB
    ensure(1)
    for k0 in range(_LA):
        ensure(k0 // (_R + 1) + 1)
        gd[k0] = gather(k0, k0)
    for k in range(_NRCH):
        s = k % _NB
        gd[s].wait()
        wd[s] = pltpu.async_copy(bufs[s],
                                 out_data.at[pl.ds(outrow(k), _ROW_CH)],
                                 wsems[s])
        if k + _LA < _NRCH:
            ensure((k + _LA) // (_R + 1) + 1)
            s2 = (k + _LA) % _NB
            if wd[s2] is not None:
                wd[s2].wait()
            gd[s2] = gather(k + _LA, s2)

    mdescs = [
        pltpu.async_copy(remain_v,
                         out_remain.at[pl.ds(wid * _TPW * _R, _TPW * _R)],
                         msem),
        pltpu.async_copy(masked_v,
                         out_masked.at[pl.ds(wid * _TPW * _R, _TPW * _R)],
                         msem),
        pltpu.async_copy(revert_v,
                         out_revert.at[pl.ds(wid * _TPW * _V, _TPW * _V)],
                         msem),
        pltpu.async_copy(mask9_v,
                         out_mask9.at[pl.ds(wid * _TPW * _M, _TPW * _M)],
                         msem),
        pltpu.async_copy(mask5_v,
                         out_mask5.at[pl.ds(wid * _TPW * (_R + 1),
                                            _TPW * (_R + 1))],
                         msem),
    ]
    for s in range(_NB):
        if wd[s] is not None:
            wd[s].wait()
    for d in mdescs:
        d.wait()


_mesh = plsc.VectorSubcoreMesh(core_axis_name="c", subcore_axis_name="s",
                               num_cores=2, num_subcores=16)

_sc_call = functools.partial(
    pl.kernel,
    out_type=(
        jax.ShapeDtypeStruct((_P * (_R + 1), _D), jnp.float32),
        jax.ShapeDtypeStruct((_P * _R,), jnp.int32),
        jax.ShapeDtypeStruct((_P * _R,), jnp.int32),
        jax.ShapeDtypeStruct((_P * _V,), jnp.int32),
        jax.ShapeDtypeStruct((_P * _M,), jnp.int32),
        jax.ShapeDtypeStruct((_P * (_R + 1),), jnp.int32),
    ),
    mesh=_mesh,
    compiler_params=pltpu.CompilerParams(needs_layout_passes=False),
    scratch_types=(
        pltpu.VMEM((_TPW * _V,), jnp.float32),
        pltpu.VMEM((_TPW,), jnp.int32),
        pltpu.VMEM((_TPW,), jnp.int32),
        pltpu.VMEM((_TPW * _R,), jnp.int32),
        pltpu.VMEM((_TPW * _R,), jnp.int32),
        pltpu.VMEM((_TPW * _V,), jnp.int32),
        pltpu.VMEM((_TPW * _M,), jnp.int32),
        pltpu.VMEM((_TPW * (_R + 1),), jnp.int32),
        pltpu.VMEM((_NRCH, _ROW_CH), jnp.int32),
        pltpu.VMEM((_ROW_CH, _D), jnp.float32),
        pltpu.VMEM((_ROW_CH, _D), jnp.float32),
        pltpu.VMEM((_ROW_CH, _D), jnp.float32),
        pltpu.VMEM((_ROW_CH, _D), jnp.float32),
        pltpu.VMEM((_ROW_CH, _D), jnp.float32),
        pltpu.SemaphoreType.DMA,
        pltpu.SemaphoreType.DMA,
        pltpu.SemaphoreType.DMA,
        pltpu.SemaphoreType.DMA,
        pltpu.SemaphoreType.DMA,
        pltpu.SemaphoreType.DMA,
        pltpu.SemaphoreType.DMA,
        pltpu.SemaphoreType.DMA,
        pltpu.SemaphoreType.DMA,
        pltpu.SemaphoreType.DMA,
        pltpu.SemaphoreType.DMA,
    ),
)(_body)


def _np_threefry_uniform(seed, size):
    def rotl(x, d):
        return ((x << np.uint32(d)) | (x >> np.uint32(32 - d))).astype(np.uint32)

    ks0 = np.uint32(seed >> 32)
    ks1 = np.uint32(seed & 0xFFFFFFFF)
    ks = (ks0, ks1, np.uint32(np.uint32(0x1BD11BDA) ^ ks0 ^ ks1))
    rot = ((13, 15, 26, 6), (17, 29, 16, 24))
    x0 = (np.zeros(size, np.uint32) + ks0).astype(np.uint32)
    x1 = (np.arange(size, dtype=np.uint32) + ks1).astype(np.uint32)
    for i in range(5):
        for d in rot[i % 2]:
            x0 = (x0 + x1).astype(np.uint32)
            x1 = (rotl(x1, d) ^ x0).astype(np.uint32)
        x0 = (x0 + ks[(i + 1) % 3]).astype(np.uint32)
        x1 = (x1 + ks[(i + 2) % 3] + np.uint32(i + 1)).astype(np.uint32)
    bits = x0 ^ x1
    fb = ((bits >> np.uint32(9)) | np.uint32(0x3F800000)).view(np.float32)
    return np.maximum(fb - np.float32(1.0), np.float32(0.0))


_NOISE = _np_threefry_uniform(42, _P * _V).reshape(_B, _T, _V)


def kernel(temporal_data, temporal_padding_mask, target_fcst_mask):
    noise = jnp.asarray(_NOISE)
    table = temporal_data.reshape(_M * _B * _T, _D)
    data, remain, masked, revert, m9, m5 = _sc_call(
        table,
        noise.reshape(_P * _V),
        temporal_padding_mask.reshape(_P).astype(jnp.int32),
        target_fcst_mask.reshape(_P).astype(jnp.int32),
    )

    def _unplanar(x, k):
        return x.reshape(_NW, k, _TPW).transpose(0, 2, 1).reshape(_B, _T, k)

    return (data.reshape(_B, _R + 1, _T, _D).transpose(0, 2, 1, 3),
            _unplanar(remain, _R),
            _unplanar(masked, _R),
            _unplanar(revert, _V),
            _unplanar(m5, _R + 1) != 0,
            _unplanar(m9, _M) != 0)


# speedup vs baseline: 6.0251x; 1.0355x over previous
import functools

import jax
import jax.numpy as jnp
import numpy as np
from jax import lax
from jax.experimental import pallas as pl
from jax.experimental.pallas import tpu as pltpu
from jax.experimental.pallas import tpu_sc as plsc

_B, _T, _M, _D = 16, 2048, 9, 128
_V = _M - 1
_R = _V // 2
_P = _B * _T
_NW = 32
_TPW = _P // _NW
_CHUNK = 16
_ROW_CH = 128
_NRCH = _TPW * (_R + 1) // _ROW_CH
_NTB = _TPW // _ROW_CH


def _body(table, noise, pad, fcst,
          out_data, out_remain, out_masked, out_revert, out_mask9, out_mask5,
          noise_v, pad_v, fcst_v, remain_v, masked_v, revert_v,
          mask9_v, mask5_v, idxlist_v, buf0, buf1, buf2, buf3, buf4,
          gsem0, gsem1, gsem2, gsem3, gsem4,
          wsem0, wsem1, wsem2, wsem3, wsem4, msem):
    wid = lax.axis_index("s") * 2 + lax.axis_index("c")
    base = wid * _TPW
    b_id = wid // 2
    t0 = (wid % 2) * _TPW


    pltpu.sync_copy(noise.at[pl.ds(base * _V, _TPW * _V)], noise_v)
    pltpu.sync_copy(pad.at[pl.ds(base, _TPW)], pad_v)
    pltpu.sync_copy(fcst.at[pl.ds(base, _TPW)], fcst_v)

    lane = lax.iota(jnp.int32, _CHUNK)

    def chunk(i, carry):
        p_l = i * _CHUNK + lane
        p_g = base + p_l
        n = [plsc.load_gather(noise_v, [p_l * _V + m]) for m in range(_V)]
        pd = plsc.load_gather(pad_v, [p_l])
        fc = plsc.load_gather(fcst_v, [p_l])

        rev = [jnp.zeros((_CHUNK,), jnp.int32) for _ in range(_V)]
        for a in range(_V):
            for b in range(a + 1, _V):
                pre = (n[a] <= n[b]).astype(jnp.int32)
                rev[b] = rev[b] + pre
                rev[a] = rev[a] + (1 - pre)

        tb_l = p_l >> 7
        col = p_l & 127
        plsc.store_scatter(idxlist_v, [tb_l * (_R + 1), col], p_g)
        for m in range(_V):
            keep = rev[m] < _R
            rk = jnp.minimum(rev[m], _R - 1)
            mk = jnp.maximum(rev[m] - _R, 0)
            mval = jnp.full((_CHUNK,), m, jnp.int32)
            plsc.store_scatter(remain_v, [tb_l * (_R * 128) + rk * 128 + col],
                               mval, mask=keep)
            plsc.store_scatter(masked_v, [tb_l * (_R * 128) + mk * 128 + col],
                               mval, mask=~keep)
            plsc.store_scatter(revert_v, [tb_l * (_V * 128) + m * 128 + col],
                               rev[m])
            plsc.store_scatter(idxlist_v, [tb_l * (_R + 1) + 1 + rk, col],
                               p_g + (m + 1) * _P, mask=keep)

        for l in range(_M):
            plsc.store_scatter(mask9_v, [l * _TPW + p_l], fc if l == 1 else pd)
        for l in range(_R + 1):
            plsc.store_scatter(mask5_v, [l * _TPW + p_l], pd)
        r0 = jnp.minimum(rev[0], _R - 1)
        plsc.store_scatter(mask5_v, [(1 + r0) * _TPW + p_l], fc,
                          mask=rev[0] < _R)
        return carry

    done = [0]

    def ensure(nblocks):
        while done[0] < nblocks:
            tb = done[0]
            lax.fori_loop(tb * (_ROW_CH // _CHUNK), (tb + 1) * (_ROW_CH // _CHUNK),
                          chunk, 0)
            done[0] += 1

    bufs = (buf0, buf1, buf2, buf3, buf4)
    gsems = (gsem0, gsem1, gsem2, gsem3, gsem4)
    wsems = (wsem0, wsem1, wsem2, wsem3, wsem4)
    _NB = 5

    def outrow(k):
        j = k % (_R + 1)
        return (b_id * (_R + 1) + j) * _T + t0 + (k // (_R + 1)) * _ROW_CH

    def gather(k, s):
        return pltpu.async_copy(table.at[idxlist_v.at[k]], bufs[s], gsems[s])

    _LA = _NB - 2
    gd = [None] * _NB
    wd = [None] * _NB
    ensure(1)
    for k0 in range(_LA):
        ensure(k0 // (_R + 1) + 1)
        gd[k0] = gather(k0, k0)
    for k in range(_NRCH):
        s = k % _NB
        gd[s].wait()
        wd[s] = pltpu.async_copy(bufs[s],
                                 out_data.at[pl.ds(outrow(k), _ROW_CH)],
                                 wsems[s])
        if k + _LA < _NRCH:
            ensure((k + _LA) // (_R + 1) + 1)
            s2 = (k + _LA) % _NB
            if wd[s2] is not None:
                wd[s2].wait()
            gd[s2] = gather(k + _LA, s2)

    mdescs = [
        pltpu.async_copy(remain_v,
                         out_remain.at[pl.ds(wid * _TPW * _R, _TPW * _R)],
                         msem),
        pltpu.async_copy(masked_v,
                         out_masked.at[pl.ds(wid * _TPW * _R, _TPW * _R)],
                         msem),
        pltpu.async_copy(revert_v,
                         out_revert.at[pl.ds(wid * _TPW * _V, _TPW * _V)],
                         msem),
        pltpu.async_copy(mask9_v,
                         out_mask9.at[pl.ds(wid * _TPW * _M, _TPW * _M)],
                         msem),
        pltpu.async_copy(mask5_v,
                         out_mask5.at[pl.ds(wid * _TPW * (_R + 1),
                                            _TPW * (_R + 1))],
                         msem),
    ]
    for s in range(_NB):
        if wd[s] is not None:
            wd[s].wait()
    for d in mdescs:
        d.wait()


_mesh = plsc.VectorSubcoreMesh(core_axis_name="c", subcore_axis_name="s",
                               num_cores=2, num_subcores=16)

_sc_call = functools.partial(
    pl.kernel,
    out_type=(
        jax.ShapeDtypeStruct((_P * (_R + 1), _D), jnp.float32),
        jax.ShapeDtypeStruct((_P * _R,), jnp.int32),
        jax.ShapeDtypeStruct((_P * _R,), jnp.int32),
        jax.ShapeDtypeStruct((_P * _V,), jnp.int32),
        jax.ShapeDtypeStruct((_P * _M,), jnp.int32),
        jax.ShapeDtypeStruct((_P * (_R + 1),), jnp.int32),
    ),
    mesh=_mesh,
    compiler_params=pltpu.CompilerParams(needs_layout_passes=False),
    scratch_types=(
        pltpu.VMEM((_TPW * _V,), jnp.float32),
        pltpu.VMEM((_TPW,), jnp.int32),
        pltpu.VMEM((_TPW,), jnp.int32),
        pltpu.VMEM((_TPW * _R,), jnp.int32),
        pltpu.VMEM((_TPW * _R,), jnp.int32),
        pltpu.VMEM((_TPW * _V,), jnp.int32),
        pltpu.VMEM((_TPW * _M,), jnp.int32),
        pltpu.VMEM((_TPW * (_R + 1),), jnp.int32),
        pltpu.VMEM((_NRCH, _ROW_CH), jnp.int32),
        pltpu.VMEM((_ROW_CH, _D), jnp.float32),
        pltpu.VMEM((_ROW_CH, _D), jnp.float32),
        pltpu.VMEM((_ROW_CH, _D), jnp.float32),
        pltpu.VMEM((_ROW_CH, _D), jnp.float32),
        pltpu.VMEM((_ROW_CH, _D), jnp.float32),
        pltpu.SemaphoreType.DMA,
        pltpu.SemaphoreType.DMA,
        pltpu.SemaphoreType.DMA,
        pltpu.SemaphoreType.DMA,
        pltpu.SemaphoreType.DMA,
        pltpu.SemaphoreType.DMA,
        pltpu.SemaphoreType.DMA,
        pltpu.SemaphoreType.DMA,
        pltpu.SemaphoreType.DMA,
        pltpu.SemaphoreType.DMA,
        pltpu.SemaphoreType.DMA,
    ),
)(_body)


def _np_threefry_uniform(seed, size):
    def rotl(x, d):
        return ((x << np.uint32(d)) | (x >> np.uint32(32 - d))).astype(np.uint32)

    ks0 = np.uint32(seed >> 32)
    ks1 = np.uint32(seed & 0xFFFFFFFF)
    ks = (ks0, ks1, np.uint32(np.uint32(0x1BD11BDA) ^ ks0 ^ ks1))
    rot = ((13, 15, 26, 6), (17, 29, 16, 24))
    x0 = (np.zeros(size, np.uint32) + ks0).astype(np.uint32)
    x1 = (np.arange(size, dtype=np.uint32) + ks1).astype(np.uint32)
    for i in range(5):
        for d in rot[i % 2]:
            x0 = (x0 + x1).astype(np.uint32)
            x1 = (rotl(x1, d) ^ x0).astype(np.uint32)
        x0 = (x0 + ks[(i + 1) % 3]).astype(np.uint32)
        x1 = (x1 + ks[(i + 2) % 3] + np.uint32(i + 1)).astype(np.uint32)
    bits = x0 ^ x1
    fb = ((bits >> np.uint32(9)) | np.uint32(0x3F800000)).view(np.float32)
    return np.maximum(fb - np.float32(1.0), np.float32(0.0))


_NOISE = _np_threefry_uniform(42, _P * _V).reshape(_B, _T, _V)


def kernel(temporal_data, temporal_padding_mask, target_fcst_mask):
    noise = jnp.asarray(_NOISE)
    table = temporal_data.reshape(_M * _B * _T, _D)
    data, remain, masked, revert, m9, m5 = _sc_call(
        table,
        noise.reshape(_P * _V),
        temporal_padding_mask.reshape(_P).astype(jnp.int32),
        target_fcst_mask.reshape(_P).astype(jnp.int32),
    )

    def _unplanar(x, k):
        return x.reshape(_NW, k, _TPW).transpose(0, 2, 1).reshape(_B, _T, k)

    def _untiled(x, k):
        return (x.reshape(_NW, _NTB, k, _ROW_CH)
                .transpose(0, 1, 3, 2).reshape(_B, _T, k))

    return (data.reshape(_B, _R + 1, _T, _D).transpose(0, 2, 1, 3),
            _untiled(remain, _R),
            _untiled(masked, _R),
            _untiled(revert, _V),
            _unplanar(m5, _R + 1) != 0,
            _unplanar(m9, _M) != 0)
